# Initial kernel scaffold; baseline (speedup 1.0000x reference)
import jax
import jax.numpy as jnp
from jax.experimental import pallas as pl


def kernel(input_data, target):
    raise NotImplementedError("write your pallas kernel here")


# speedup vs baseline: 1.1117x; 1.1117x over previous
import functools

import jax
import jax.numpy as jnp
from jax import lax
from jax.experimental import pallas as pl
from jax.experimental.pallas import tpu as pltpu
from jax.experimental.pallas import tpu_sc as plsc

_B, _C, _T = 1024, 1000, 20
_E = _B * _T
_NW = 32
_CHUNK = 128
_NCHUNK = _E // _CHUNK
_KPW = _NCHUNK // _NW
_L = 16


def _sc_gather_body(tgt_hbm, xflat_hbm, out_hbm, idx_v, vals_v, sem):
    wid = lax.axis_index("s") * 2 + lax.axis_index("c")

    pltpu.sync_copy(tgt_hbm.at[wid], idx_v)

    lane = lax.iota(jnp.int32, _L)
    for k in range(_KPW):
        for s in range(_CHUNK // _L):
            off = s * _L
            t = idx_v[k, pl.ds(off, _L)]
            e0 = (wid * _KPW + k) * _CHUNK + off
            b = lax.div(e0 + lane, jnp.int32(_T))
            idx_v[k, pl.ds(off, _L)] = b * _C + jnp.clip(t, 0, _C - 1)

    copies = [
        pltpu.async_copy(xflat_hbm.at[idx_v.at[k]], vals_v.at[k], sem)
        for k in range(_KPW)
    ]
    for c in copies:
        c.wait()

    pltpu.sync_copy(vals_v, out_hbm.at[wid])


def _sc_gather(tgt2d, x_flat):
    f = pl.kernel(
        _sc_gather_body,
        mesh=plsc.VectorSubcoreMesh(core_axis_name="c", subcore_axis_name="s"),
        out_type=jax.ShapeDtypeStruct((_NW, _KPW, _CHUNK), jnp.float32),
        scratch_types=[
            pltpu.VMEM((_KPW, _CHUNK), jnp.int32),
            pltpu.VMEM((_KPW, _CHUNK), jnp.float32),
            pltpu.SemaphoreType.DMA,
        ],
    )
    return f(tgt2d, x_flat)


def _tc_combine_body(x_ref, g_ref, out_ref):
    s = jnp.sum(jnp.exp(-x_ref[...]), axis=1, keepdims=True)
    g = jnp.sum(jnp.exp(g_ref[...]), axis=1, keepdims=True)
    loss_data = jnp.sum(s * g) - jnp.float32(_E)
    out_ref[0, 0] = jnp.log(1.0 + loss_data) / _C


def kernel(input_data, target):
    tgt3d = target.reshape(_NW, _KPW, _CHUNK)
    x_flat = input_data.reshape(_B * _C)
    vals = _sc_gather(tgt3d, x_flat)
    out = pl.pallas_call(
        _tc_combine_body,
        out_shape=jax.ShapeDtypeStruct((1, 1), jnp.float32),
        out_specs=pl.BlockSpec(memory_space=pltpu.SMEM),
    )(input_data, vals.reshape(_B, _T))
    return out[0, 0]
